# Initial kernel scaffold; baseline (speedup 1.0000x reference)
import jax
import jax.numpy as jnp
from jax.experimental import pallas as pl


def kernel(des, tweet, num_prop, cat_prop, edge_index, edge_type, W_des, b_des, W_tw, b_tw, W_np, b_np, W_cp, b_cp, W_in, b_in, W_rel, W_root, b_rgcn, W_o1, b_o1, W_o2, b_o2):
    raise NotImplementedError("write your pallas kernel here")


# speedup vs baseline: 2.8122x; 2.8122x over previous
import functools

import jax
import jax.numpy as jnp
from jax import lax
from jax.experimental import pallas as pl
from jax.experimental.pallas import tpu as pltpu
from jax.experimental.pallas import tpu_sc as plsc

_NC = 2
_NS = 16
_L = 16
_R = 5
_D = 128
_CHK = 80


def _mesh():
    return plsc.VectorSubcoreMesh(core_axis_name="c", subcore_axis_name="s")


def _cnt_pass(et, dst, oh, zeros_n16):
    E = et.shape[0]
    N = zeros_n16.shape[0]
    NW = _NC * _NS
    epw = E // NW
    nit = epw // _CHK
    rpt = N // _NS
    crow = 128
    cpt = rpt // crow

    @functools.partial(
        pl.kernel,
        out_type=jax.ShapeDtypeStruct((_NC, N, _D), jnp.float32),
        mesh=_mesh(),
        scratch_types=[
            pltpu.VMEM_SHARED((N, _D), jnp.float32),
            pltpu.VMEM((_CHK,), jnp.int32),
            pltpu.VMEM((_CHK,), jnp.int32),
            pltpu.VMEM((_CHK, _D), jnp.float32),
            pltpu.VMEM((crow, _D), jnp.float32),
            pltpu.SemaphoreType.DMA,
        ],
    )
    def k(et_hbm, dst_hbm, oh_hbm, z_hbm, out_hbm, acc, tv, dv, ohrows, obuf, sem):
        c = lax.axis_index("c")
        s = lax.axis_index("s")
        wid = s * _NC + c
        rowbase = s * rpt
        pltpu.sync_copy(z_hbm.at[pl.ds(rowbase, rpt)], acc.at[pl.ds(rowbase, rpt)])
        plsc.subcore_barrier()
        ebase = wid * epw

        def body(i, carry):
            off = ebase + i * _CHK
            pltpu.sync_copy(et_hbm.at[pl.ds(off, _CHK)], tv)
            pltpu.sync_copy(dst_hbm.at[pl.ds(off, _CHK)], dv)
            pltpu.async_copy(oh_hbm.at[tv], ohrows, sem).wait()
            pltpu.sync_copy(ohrows, acc.at[dv], add=True)
            return carry

        lax.fori_loop(0, nit, body, 0)
        plsc.subcore_barrier()
        for j in range(cpt):
            rb = rowbase + j * crow
            pltpu.sync_copy(acc.at[pl.ds(rb, crow)], obuf)
            pltpu.sync_copy(obuf, out_hbm.at[c, pl.ds(rb, crow)])

    return k(et, dst, oh, zeros_n16)


def _weight_pass(src, dst, et, rtflat):
    E = src.shape[0]
    N = rtflat.shape[0] // _R
    NW = _NC * _NS
    epw = E // NW
    nit = epw // _CHK

    @functools.partial(
        pl.kernel,
        out_type=[
            jax.ShapeDtypeStruct((E,), jnp.int32),
            jax.ShapeDtypeStruct((E * _L,), jnp.float32),
        ],
        mesh=_mesh(),
        scratch_types=[
            pltpu.VMEM((_CHK,), jnp.int32),
            pltpu.VMEM((_CHK,), jnp.int32),
            pltpu.VMEM((_CHK,), jnp.int32),
            pltpu.VMEM((_CHK, _D), jnp.float32),
            pltpu.VMEM((_CHK,), jnp.int32),
            pltpu.VMEM((_CHK,), jnp.int32),
            pltpu.VMEM((_CHK * _L,), jnp.float32),
            pltpu.SemaphoreType.DMA,
        ],
    )
    def k(src_hbm, dst_hbm, et_hbm, rt_hbm, g_hbm, w_hbm,
          sv, dv, tv, rr, gb, g2, wb, sem):
        c = lax.axis_index("c")
        s = lax.axis_index("s")
        wid = s * _NC + c
        ebase = wid * epw

        def body(i, carry):
            off = ebase + i * _CHK
            pltpu.sync_copy(src_hbm.at[pl.ds(off, _CHK)], sv)
            pltpu.sync_copy(dst_hbm.at[pl.ds(off, _CHK)], dv)
            pltpu.sync_copy(et_hbm.at[pl.ds(off, _CHK)], tv)
            for j in range(_CHK // _L):
                sl = pl.ds(j * _L, _L)
                t16 = tv[sl]
                gb[sl] = t16 * N + sv[sl]
                g2[sl] = t16 * N + dv[sl]
            pltpu.async_copy(rt_hbm.at[g2], rr, sem).wait()
            for j in range(_CHK):
                wb[pl.ds(j * _L, _L)] = rr[j, pl.ds(0, _L)]
            pltpu.sync_copy(gb, g_hbm.at[pl.ds(off, _CHK)])
            pltpu.sync_copy(wb, w_hbm.at[pl.ds(off * _L, _CHK * _L)])
            return carry

        lax.fori_loop(0, nit, body, 0)

    return k(src, dst, et, rtflat)


def _edge_pass(yflat, gidx, dst, w, zeros_nd):
    E = gidx.shape[0]
    N = zeros_nd.shape[0]
    NW = _NC * _NS
    epw = E // NW
    nit = epw // _CHK
    rpt = N // _NS
    crow = 128
    cpt = rpt // crow

    @functools.partial(
        pl.kernel,
        out_type=jax.ShapeDtypeStruct((_NC, N, _D), jnp.float32),
        mesh=_mesh(),
        scratch_types=[
            pltpu.VMEM_SHARED((N, _D), jnp.float32),
            pltpu.VMEM((_CHK,), jnp.int32),
            pltpu.VMEM((_CHK,), jnp.int32),
            pltpu.VMEM((_CHK * _L,), jnp.float32),
            pltpu.VMEM((_CHK, _D), jnp.float32),
            pltpu.VMEM((crow, _D), jnp.float32),
            pltpu.SemaphoreType.DMA,
        ],
    )
    def k(y_hbm, g_hbm, d_hbm, w_hbm, z_hbm, out_hbm,
          acc, gv, dv, wv, rows, obuf, sem):
        c = lax.axis_index("c")
        s = lax.axis_index("s")
        wid = s * _NC + c
        rowbase = s * rpt
        pltpu.sync_copy(z_hbm.at[pl.ds(rowbase, rpt)], acc.at[pl.ds(rowbase, rpt)])
        plsc.subcore_barrier()
        ebase = wid * epw

        def body(i, carry):
            off = ebase + i * _CHK
            pltpu.sync_copy(g_hbm.at[pl.ds(off, _CHK)], gv)
            pltpu.sync_copy(d_hbm.at[pl.ds(off, _CHK)], dv)
            pltpu.sync_copy(w_hbm.at[pl.ds(off * _L, _CHK * _L)], wv)
            pltpu.async_copy(y_hbm.at[gv], rows, sem).wait()

            def scale(j, cc):
                ws = wv[pl.ds(j * _L, _L)]
                for kk in range(_D // _L):
                    sl = pl.ds(kk * _L, _L)
                    rows[j, sl] = rows[j, sl] * ws
                return cc

            lax.fori_loop(0, _CHK, scale, 0)
            pltpu.sync_copy(rows, acc.at[dv], add=True)
            return carry

        lax.fori_loop(0, nit, body, 0)
        plsc.subcore_barrier()
        for j in range(cpt):
            rb = rowbase + j * crow
            pltpu.sync_copy(acc.at[pl.ds(rb, crow)], obuf)
            pltpu.sync_copy(obuf, out_hbm.at[c, pl.ds(rb, crow)])

    return k(yflat, gidx, dst, w, zeros_nd)


def _lrelu(v):
    return jnp.where(v >= 0, v, 0.01 * v)


def _front(des, tweet, num_prop, cat_prop, cntp,
           W_des, b_des, W_tw, b_tw, W_np, b_np, W_cp, b_cp,
           W_in, b_in, W_rel, W_root, b_rgcn):
    N = des.shape[0]
    B = 1000
    G = N // B

    def body(des_r, tw_r, np_r, cp_r, cnt_r,
             wdes_r, bdes_r, wtw_r, btw_r, wnp_r, bnp_r, wcp_r, bcp_r,
             win_r, bin_r, wrel_r, wroot_r, brg_r,
             y_r, root_r, rcp_r):
        d = _lrelu(jnp.dot(des_r[...], wdes_r[...],
                           preferred_element_type=jnp.float32) + bdes_r[...])
        t = _lrelu(jnp.dot(tw_r[...], wtw_r[...],
                           preferred_element_type=jnp.float32) + btw_r[...])
        n = _lrelu(jnp.dot(np_r[...], wnp_r[...],
                           preferred_element_type=jnp.float32) + bnp_r[...])
        cc = _lrelu(jnp.dot(cp_r[...], wcp_r[...],
                            preferred_element_type=jnp.float32) + bcp_r[...])
        x = jnp.concatenate([d, t, n, cc], axis=1)
        x1 = _lrelu(jnp.dot(x, win_r[...],
                            preferred_element_type=jnp.float32) + bin_r[...])
        root_r[...] = jnp.dot(x1, wroot_r[...],
                              preferred_element_type=jnp.float32) + brg_r[...]
        for r in range(_R):
            y_r[r] = jnp.dot(x1, wrel_r[r],
                             preferred_element_type=jnp.float32)
        cnt = cnt_r[0] + cnt_r[1]
        rc = 1.0 / jnp.maximum(cnt, 1.0)
        for r in range(_R):
            rcp_r[r] = jnp.broadcast_to(rc[:, r:r + 1], rc.shape)

    full2 = lambda a: pl.BlockSpec(a.shape, lambda i: (0,) * a.ndim)
    return pl.pallas_call(
        body,
        grid=(G,),
        in_specs=[
            pl.BlockSpec((B, 768), lambda i: (i, 0)),
            pl.BlockSpec((B, 768), lambda i: (i, 0)),
            pl.BlockSpec((B, 6), lambda i: (i, 0)),
            pl.BlockSpec((B, 11), lambda i: (i, 0)),
            pl.BlockSpec((_NC, B, _D), lambda i: (0, i, 0)),
            full2(W_des), full2(b_des), full2(W_tw), full2(b_tw),
            full2(W_np), full2(b_np), full2(W_cp), full2(b_cp),
            full2(W_in), full2(b_in), full2(W_rel), full2(W_root),
            full2(b_rgcn),
        ],
        out_specs=[
            pl.BlockSpec((_R, B, _D), lambda i: (0, i, 0)),
            pl.BlockSpec((B, _D), lambda i: (i, 0)),
            pl.BlockSpec((_R, B, _D), lambda i: (0, i, 0)),
        ],
        out_shape=[
            jax.ShapeDtypeStruct((_R, N, _D), jnp.float32),
            jax.ShapeDtypeStruct((N, _D), jnp.float32),
            jax.ShapeDtypeStruct((_R, N, _D), jnp.float32),
        ],
    )(des, tweet, num_prop, cat_prop, cntp,
      W_des, b_des, W_tw, b_tw, W_np, b_np, W_cp, b_cp,
      W_in, b_in, W_rel, W_root, b_rgcn)


def _mid(root1, parts, W_rel, W_root, b_rgcn):
    N = root1.shape[0]
    B = 1000
    G = N // B

    def body(root_r, p_r, wrel_r, wroot_r, brg_r, y_r, root2_r):
        h = root_r[...] + p_r[0] + p_r[1]
        root2_r[...] = jnp.dot(h, wroot_r[...],
                               preferred_element_type=jnp.float32) + brg_r[...]
        for r in range(_R):
            y_r[r] = jnp.dot(h, wrel_r[r], preferred_element_type=jnp.float32)

    full2 = lambda a: pl.BlockSpec(a.shape, lambda i: (0,) * a.ndim)
    return pl.pallas_call(
        body,
        grid=(G,),
        in_specs=[
            pl.BlockSpec((B, _D), lambda i: (i, 0)),
            pl.BlockSpec((_NC, B, _D), lambda i: (0, i, 0)),
            full2(W_rel), full2(W_root), full2(b_rgcn),
        ],
        out_specs=[
            pl.BlockSpec((_R, B, _D), lambda i: (0, i, 0)),
            pl.BlockSpec((B, _D), lambda i: (i, 0)),
        ],
        out_shape=[
            jax.ShapeDtypeStruct((_R, N, _D), jnp.float32),
            jax.ShapeDtypeStruct((N, _D), jnp.float32),
        ],
    )(root1, parts, W_rel, W_root, b_rgcn)


def _head(root2, parts, W_o1, b_o1, W_o2, b_o2):
    N = root2.shape[0]
    B = 1000
    G = N // B

    def body(root_r, p_r, w1_r, b1_r, w2_r, b2_r, out_r):
        h = root_r[...] + p_r[0] + p_r[1]
        x = _lrelu(jnp.dot(h, w1_r[...],
                           preferred_element_type=jnp.float32) + b1_r[...])
        out_r[...] = jnp.dot(x, w2_r[...],
                             preferred_element_type=jnp.float32) + b2_r[...]

    full2 = lambda a: pl.BlockSpec(a.shape, lambda i: (0,) * a.ndim)
    return pl.pallas_call(
        body,
        grid=(G,),
        in_specs=[
            pl.BlockSpec((B, _D), lambda i: (i, 0)),
            pl.BlockSpec((_NC, B, _D), lambda i: (0, i, 0)),
            full2(W_o1), full2(b_o1), full2(W_o2), full2(b_o2),
        ],
        out_specs=pl.BlockSpec((B, 2), lambda i: (i, 0)),
        out_shape=jax.ShapeDtypeStruct((N, 2), jnp.float32),
    )(root2, parts, W_o1, b_o1, W_o2, b_o2)


def kernel(des, tweet, num_prop, cat_prop, edge_index, edge_type,
           W_des, b_des, W_tw, b_tw, W_np, b_np, W_cp, b_cp,
           W_in, b_in, W_rel, W_root, b_rgcn, W_o1, b_o1, W_o2, b_o2):
    N = des.shape[0]
    src = edge_index[0].astype(jnp.int32)
    dst = edge_index[1].astype(jnp.int32)
    et = edge_type.astype(jnp.int32)

    npad = ((N + 16 * 8 * _NS - 1) // (16 * 8 * _NS)) * (16 * 8 * _NS)
    oh = jnp.eye(_R, _D, dtype=jnp.float32)
    zeros_nd = jnp.zeros((npad, _D), jnp.float32)
    r2 = lambda b: b.reshape(1, -1)

    cntp = _cnt_pass(et, dst, oh, zeros_nd)
    Y1, root1, rt = _front(
        des, tweet, num_prop, cat_prop, cntp,
        W_des, r2(b_des), W_tw, r2(b_tw), W_np, r2(b_np), W_cp, r2(b_cp),
        W_in, r2(b_in), W_rel, W_root, r2(b_rgcn))
    gidx, w = _weight_pass(src, dst, et, rt.reshape(_R * N, _D))
    p1 = _edge_pass(Y1.reshape(_R * N, _D), gidx, dst, w, zeros_nd)
    Y2, root2 = _mid(root1, p1, W_rel, W_root, r2(b_rgcn))
    p2 = _edge_pass(Y2.reshape(_R * N, _D), gidx, dst, w, zeros_nd)
    return _head(root2, p2, W_o1, r2(b_o1), W_o2, r2(b_o2))


# speedup vs baseline: 6.3442x; 2.2559x over previous
import functools

import jax
import jax.numpy as jnp
from jax import lax
from jax.experimental import pallas as pl
from jax.experimental.pallas import tpu as pltpu
from jax.experimental.pallas import tpu_sc as plsc

_NC = 2
_NS = 16
_L = 16
_R = 5
_D = 128
_CHK = 80


def _mesh():
    return plsc.VectorSubcoreMesh(core_axis_name="c", subcore_axis_name="s")


def _cnt_pass(et, dst, oh, zeros_n16):
    E = et.shape[0]
    N = zeros_n16.shape[0]
    NW = _NC * _NS
    epw = E // NW
    nit = epw // _CHK
    rpt = N // _NS
    crow = 128
    cpt = rpt // crow

    @functools.partial(
        pl.kernel,
        out_type=jax.ShapeDtypeStruct((_NC, N, _D), jnp.float32),
        mesh=_mesh(),
        scratch_types=[
            pltpu.VMEM_SHARED((N, _D), jnp.float32),
            pltpu.VMEM((_CHK,), jnp.int32),
            pltpu.VMEM((_CHK,), jnp.int32),
            pltpu.VMEM((_CHK, _D), jnp.float32),
            pltpu.VMEM((crow, _D), jnp.float32),
            pltpu.SemaphoreType.DMA,
        ],
    )
    def k(et_hbm, dst_hbm, oh_hbm, z_hbm, out_hbm, acc, tv, dv, ohrows, obuf, sem):
        c = lax.axis_index("c")
        s = lax.axis_index("s")
        wid = s * _NC + c
        rowbase = s * rpt
        pltpu.sync_copy(z_hbm.at[pl.ds(rowbase, rpt)], acc.at[pl.ds(rowbase, rpt)])
        plsc.subcore_barrier()
        ebase = wid * epw
        nrep = oh_hbm.shape[0] // _R

        def body(i, carry):
            off = ebase + i * _CHK
            pltpu.sync_copy(et_hbm.at[pl.ds(off, _CHK)], tv)
            pltpu.sync_copy(dst_hbm.at[pl.ds(off, _CHK)], dv)
            rot = (i * (_CHK // _L) + wid) * 7
            for j in range(_CHK // _L):
                sl = pl.ds(j * _L, _L)
                m = lax.rem(lax.iota(jnp.int32, _L) * 3 + rot + j, nrep)
                tv[sl] = tv[sl] + m * _R
            pltpu.async_copy(oh_hbm.at[tv], ohrows, sem).wait()
            pltpu.sync_copy(ohrows, acc.at[dv], add=True)
            return carry

        lax.fori_loop(0, nit, body, 0)
        plsc.subcore_barrier()
        for j in range(cpt):
            rb = rowbase + j * crow
            pltpu.sync_copy(acc.at[pl.ds(rb, crow)], obuf)
            pltpu.sync_copy(obuf, out_hbm.at[c, pl.ds(rb, crow)])

    return k(et, dst, oh, zeros_n16)


def _weight_pass(src, dst, et, rtflat):
    E = src.shape[0]
    N = rtflat.shape[0] // _R
    NW = _NC * _NS
    epw = E // NW
    nit = epw // _CHK

    @functools.partial(
        pl.kernel,
        out_type=[
            jax.ShapeDtypeStruct((E,), jnp.int32),
            jax.ShapeDtypeStruct((E * _L,), jnp.float32),
        ],
        mesh=_mesh(),
        scratch_types=[
            pltpu.VMEM((_CHK,), jnp.int32),
            pltpu.VMEM((_CHK,), jnp.int32),
            pltpu.VMEM((_CHK,), jnp.int32),
            pltpu.VMEM((_CHK, _D), jnp.float32),
            pltpu.VMEM((_CHK,), jnp.int32),
            pltpu.VMEM((_CHK,), jnp.int32),
            pltpu.VMEM((_CHK * _L,), jnp.float32),
            pltpu.SemaphoreType.DMA,
        ],
    )
    def k(src_hbm, dst_hbm, et_hbm, rt_hbm, g_hbm, w_hbm,
          sv, dv, tv, rr, gb, g2, wb, sem):
        c = lax.axis_index("c")
        s = lax.axis_index("s")
        wid = s * _NC + c
        ebase = wid * epw

        def body(i, carry):
            off = ebase + i * _CHK
            pltpu.sync_copy(src_hbm.at[pl.ds(off, _CHK)], sv)
            pltpu.sync_copy(dst_hbm.at[pl.ds(off, _CHK)], dv)
            pltpu.sync_copy(et_hbm.at[pl.ds(off, _CHK)], tv)
            for j in range(_CHK // _L):
                sl = pl.ds(j * _L, _L)
                t16 = tv[sl]
                gb[sl] = t16 * N + sv[sl]
                g2[sl] = t16 * N + dv[sl]
            pltpu.async_copy(rt_hbm.at[g2], rr, sem).wait()
            for j in range(_CHK):
                wb[pl.ds(j * _L, _L)] = rr[j, pl.ds(0, _L)]
            pltpu.sync_copy(gb, g_hbm.at[pl.ds(off, _CHK)])
            pltpu.sync_copy(wb, w_hbm.at[pl.ds(off * _L, _CHK * _L)])
            return carry

        lax.fori_loop(0, nit, body, 0)

    return k(src, dst, et, rtflat)


def _edge_pass(yflat, gidx, dst, w, zeros_nd):
    E = gidx.shape[0]
    N = zeros_nd.shape[0]
    NW = _NC * _NS
    epw = E // NW
    nit = epw // _CHK
    rpt = N // _NS
    crow = 128
    cpt = rpt // crow

    @functools.partial(
        pl.kernel,
        out_type=jax.ShapeDtypeStruct((_NC, N, _D), jnp.float32),
        mesh=_mesh(),
        scratch_types=[
            pltpu.VMEM_SHARED((N, _D), jnp.float32),
            pltpu.VMEM((_CHK,), jnp.int32),
            pltpu.VMEM((_CHK,), jnp.int32),
            pltpu.VMEM((_CHK * _L,), jnp.float32),
            pltpu.VMEM((_CHK, _D), jnp.float32),
            pltpu.VMEM((crow, _D), jnp.float32),
            pltpu.SemaphoreType.DMA,
        ],
    )
    def k(y_hbm, g_hbm, d_hbm, w_hbm, z_hbm, out_hbm,
          acc, gv, dv, wv, rows, obuf, sem):
        c = lax.axis_index("c")
        s = lax.axis_index("s")
        wid = s * _NC + c
        rowbase = s * rpt
        pltpu.sync_copy(z_hbm.at[pl.ds(rowbase, rpt)], acc.at[pl.ds(rowbase, rpt)])
        plsc.subcore_barrier()
        ebase = wid * epw

        def body(i, carry):
            off = ebase + i * _CHK
            pltpu.sync_copy(g_hbm.at[pl.ds(off, _CHK)], gv)
            pltpu.sync_copy(d_hbm.at[pl.ds(off, _CHK)], dv)
            pltpu.sync_copy(w_hbm.at[pl.ds(off * _L, _CHK * _L)], wv)
            pltpu.async_copy(y_hbm.at[gv], rows, sem).wait()

            def scale(j, cc):
                ws = wv[pl.ds(j * _L, _L)]
                for kk in range(_D // _L):
                    sl = pl.ds(kk * _L, _L)
                    rows[j, sl] = rows[j, sl] * ws
                return cc

            lax.fori_loop(0, _CHK, scale, 0)
            pltpu.sync_copy(rows, acc.at[dv], add=True)
            return carry

        lax.fori_loop(0, nit, body, 0)
        plsc.subcore_barrier()
        for j in range(cpt):
            rb = rowbase + j * crow
            pltpu.sync_copy(acc.at[pl.ds(rb, crow)], obuf)
            pltpu.sync_copy(obuf, out_hbm.at[c, pl.ds(rb, crow)])

    return k(yflat, gidx, dst, w, zeros_nd)


def _lrelu(v):
    return jnp.where(v >= 0, v, 0.01 * v)


def _front(des, tweet, num_prop, cat_prop, cntp,
           W_des, b_des, W_tw, b_tw, W_np, b_np, W_cp, b_cp,
           W_in, b_in, W_rel, W_root, b_rgcn):
    N = des.shape[0]
    B = 1000
    G = N // B

    def body(des_r, tw_r, np_r, cp_r, cnt_r,
             wdes_r, bdes_r, wtw_r, btw_r, wnp_r, bnp_r, wcp_r, bcp_r,
             win_r, bin_r, wrel_r, wroot_r, brg_r,
             y_r, root_r, rcp_r):
        d = _lrelu(jnp.dot(des_r[...], wdes_r[...],
                           preferred_element_type=jnp.float32) + bdes_r[...])
        t = _lrelu(jnp.dot(tw_r[...], wtw_r[...],
                           preferred_element_type=jnp.float32) + btw_r[...])
        n = _lrelu(jnp.dot(np_r[...], wnp_r[...],
                           preferred_element_type=jnp.float32) + bnp_r[...])
        cc = _lrelu(jnp.dot(cp_r[...], wcp_r[...],
                            preferred_element_type=jnp.float32) + bcp_r[...])
        x = jnp.concatenate([d, t, n, cc], axis=1)
        x1 = _lrelu(jnp.dot(x, win_r[...],
                            preferred_element_type=jnp.float32) + bin_r[...])
        root_r[...] = jnp.dot(x1, wroot_r[...],
                              preferred_element_type=jnp.float32) + brg_r[...]
        for r in range(_R):
            y_r[r] = jnp.dot(x1, wrel_r[r],
                             preferred_element_type=jnp.float32)
        cnt = cnt_r[0] + cnt_r[1]
        rc = 1.0 / jnp.maximum(cnt, 1.0)
        for r in range(_R):
            rcp_r[r] = jnp.broadcast_to(rc[:, r:r + 1], rc.shape)

    full2 = lambda a: pl.BlockSpec(a.shape, lambda i: (0,) * a.ndim)
    return pl.pallas_call(
        body,
        grid=(G,),
        in_specs=[
            pl.BlockSpec((B, 768), lambda i: (i, 0)),
            pl.BlockSpec((B, 768), lambda i: (i, 0)),
            pl.BlockSpec((B, 6), lambda i: (i, 0)),
            pl.BlockSpec((B, 11), lambda i: (i, 0)),
            pl.BlockSpec((_NC, B, _D), lambda i: (0, i, 0)),
            full2(W_des), full2(b_des), full2(W_tw), full2(b_tw),
            full2(W_np), full2(b_np), full2(W_cp), full2(b_cp),
            full2(W_in), full2(b_in), full2(W_rel), full2(W_root),
            full2(b_rgcn),
        ],
        out_specs=[
            pl.BlockSpec((_R, B, _D), lambda i: (0, i, 0)),
            pl.BlockSpec((B, _D), lambda i: (i, 0)),
            pl.BlockSpec((_R, B, _D), lambda i: (0, i, 0)),
        ],
        out_shape=[
            jax.ShapeDtypeStruct((_R, N, _D), jnp.float32),
            jax.ShapeDtypeStruct((N, _D), jnp.float32),
            jax.ShapeDtypeStruct((_R, N, _D), jnp.float32),
        ],
    )(des, tweet, num_prop, cat_prop, cntp,
      W_des, b_des, W_tw, b_tw, W_np, b_np, W_cp, b_cp,
      W_in, b_in, W_rel, W_root, b_rgcn)


def _mid(root1, parts, W_rel, W_root, b_rgcn):
    N = root1.shape[0]
    B = 1000
    G = N // B

    def body(root_r, p_r, wrel_r, wroot_r, brg_r, y_r, root2_r):
        h = root_r[...] + p_r[0] + p_r[1]
        root2_r[...] = jnp.dot(h, wroot_r[...],
                               preferred_element_type=jnp.float32) + brg_r[...]
        for r in range(_R):
            y_r[r] = jnp.dot(h, wrel_r[r], preferred_element_type=jnp.float32)

    full2 = lambda a: pl.BlockSpec(a.shape, lambda i: (0,) * a.ndim)
    return pl.pallas_call(
        body,
        grid=(G,),
        in_specs=[
            pl.BlockSpec((B, _D), lambda i: (i, 0)),
            pl.BlockSpec((_NC, B, _D), lambda i: (0, i, 0)),
            full2(W_rel), full2(W_root), full2(b_rgcn),
        ],
        out_specs=[
            pl.BlockSpec((_R, B, _D), lambda i: (0, i, 0)),
            pl.BlockSpec((B, _D), lambda i: (i, 0)),
        ],
        out_shape=[
            jax.ShapeDtypeStruct((_R, N, _D), jnp.float32),
            jax.ShapeDtypeStruct((N, _D), jnp.float32),
        ],
    )(root1, parts, W_rel, W_root, b_rgcn)


def _head(root2, parts, W_o1, b_o1, W_o2, b_o2):
    N = root2.shape[0]
    B = 1000
    G = N // B

    def body(root_r, p_r, w1_r, b1_r, w2_r, b2_r, out_r):
        h = root_r[...] + p_r[0] + p_r[1]
        x = _lrelu(jnp.dot(h, w1_r[...],
                           preferred_element_type=jnp.float32) + b1_r[...])
        out_r[...] = jnp.dot(x, w2_r[...],
                             preferred_element_type=jnp.float32) + b2_r[...]

    full2 = lambda a: pl.BlockSpec(a.shape, lambda i: (0,) * a.ndim)
    return pl.pallas_call(
        body,
        grid=(G,),
        in_specs=[
            pl.BlockSpec((B, _D), lambda i: (i, 0)),
            pl.BlockSpec((_NC, B, _D), lambda i: (0, i, 0)),
            full2(W_o1), full2(b_o1), full2(W_o2), full2(b_o2),
        ],
        out_specs=pl.BlockSpec((B, 2), lambda i: (i, 0)),
        out_shape=jax.ShapeDtypeStruct((N, 2), jnp.float32),
    )(root2, parts, W_o1, b_o1, W_o2, b_o2)


def kernel(des, tweet, num_prop, cat_prop, edge_index, edge_type,
           W_des, b_des, W_tw, b_tw, W_np, b_np, W_cp, b_cp,
           W_in, b_in, W_rel, W_root, b_rgcn, W_o1, b_o1, W_o2, b_o2):
    N = des.shape[0]
    src = edge_index[0].astype(jnp.int32)
    dst = edge_index[1].astype(jnp.int32)
    et = edge_type.astype(jnp.int32)

    npad = ((N + 16 * 8 * _NS - 1) // (16 * 8 * _NS)) * (16 * 8 * _NS)
    oh = jnp.tile(jnp.eye(_R, _D, dtype=jnp.float32), (256, 1))
    zeros_nd = jnp.zeros((npad, _D), jnp.float32)
    r2 = lambda b: b.reshape(1, -1)

    cntp = _cnt_pass(et, dst, oh, zeros_nd)
    Y1, root1, rt = _front(
        des, tweet, num_prop, cat_prop, cntp,
        W_des, r2(b_des), W_tw, r2(b_tw), W_np, r2(b_np), W_cp, r2(b_cp),
        W_in, r2(b_in), W_rel, W_root, r2(b_rgcn))
    gidx, w = _weight_pass(src, dst, et, rt.reshape(_R * N, _D))
    p1 = _edge_pass(Y1.reshape(_R * N, _D), gidx, dst, w, zeros_nd)
    Y2, root2 = _mid(root1, p1, W_rel, W_root, r2(b_rgcn))
    p2 = _edge_pass(Y2.reshape(_R * N, _D), gidx, dst, w, zeros_nd)
    return _head(root2, p2, W_o1, r2(b_o1), W_o2, r2(b_o2))
